# baseline (speedup 1.0000x reference)
import numpy as np

import jax
import jax.numpy as jnp
from jax.experimental import pallas as pl
from jax.experimental.pallas import tpu as pltpu

_C = 8
_L = 16
_N = 128
_ROUNDS = 7

_SHIFTS = np.zeros((_ROUNDS, _N, _N), np.float32)
for _i in range(_ROUNDS):
    _d = 1 << _i
    _SHIFTS[_i, np.arange(_N - _d), np.arange(_d, _N)] = 1.0


def _tp(a, b):
    A, n = a.shape
    Bd = b.shape[0]
    return (a[:, None, :] * b[None, :, :]).reshape(A * Bd, n)


def _exp1(d):
    e2 = _tp(d * 0.5, d)
    e3 = _tp(e2, d * (1.0 / 3.0))
    e4 = _tp(e3, d * 0.25)
    return (d, e2, e3, e4)


def _mul3(a, b):
    a1, a2, a3 = a[0], a[1], a[2]
    b1, b2, b3 = b[0], b[1], b[2]
    parts = []
    for m in range(_C):
        parts.append(a3[m * 64:(m + 1) * 64] + b3[m * 64:(m + 1) * 64]
                     + a1[m:m + 1] * b2
                     + _tp(a2[m * 8:(m + 1) * 8], b1))
    return jnp.concatenate(parts, axis=0)


def _mul4(a, b):
    a1, a2, a3, a4 = a
    b1, b2, b3, b4 = b
    parts = []
    for m in range(_C):
        parts.append(a4[m * 512:(m + 1) * 512] + b4[m * 512:(m + 1) * 512]
                     + a1[m:m + 1] * b3
                     + _tp(a2[m * 8:(m + 1) * 8], b2)
                     + _tp(a3[m * 64:(m + 1) * 64], b1))
    return jnp.concatenate(parts, axis=0)


def _ta_mul(a, b):
    c1 = a[0] + b[0]
    c2 = a[1] + b[1] + _tp(a[0], b[0])
    c3 = _mul3(a, b)
    c4 = _mul4(a, b)
    return (c1, c2, c3, c4)


def _chen_step(carry, d):
    a1, a2, a3, a4 = carry
    e2 = _tp(d * 0.5, d)
    e3 = _tp(e2, d * (1.0 / 3.0))
    dq = d * 0.25
    c1 = a1 + d
    c2 = a2 + e2 + _tp(a1, d)
    p3 = []
    p4 = []
    for m in range(_C):
        p3.append(a3[m * 64:(m + 1) * 64] + e3[m * 64:(m + 1) * 64]
                  + a1[m:m + 1] * e2
                  + _tp(a2[m * 8:(m + 1) * 8], d))
        p4.append(a4[m * 512:(m + 1) * 512]
                  + _tp(e3[m * 64:(m + 1) * 64], dq)
                  + a1[m:m + 1] * e3
                  + _tp(a2[m * 8:(m + 1) * 8], e2)
                  + _tp(a3[m * 64:(m + 1) * 64], d))
    return (c1, c2, jnp.concatenate(p3, axis=0), jnp.concatenate(p4, axis=0))


def _blamem_kernel(inc_ref, sh_ref, w1_ref, b1_ref, w2_ref, b2_ref, out_ref):
    carry0 = _exp1(inc_ref[0, 0])

    def chen_body(s, carry):
        d = inc_ref[0, s]
        return _chen_step(carry, d)

    sig = jax.lax.fori_loop(1, _L, chen_body, carry0)

    def scan_body(i, pref):
        s_mat = sh_ref[i]
        shifted = tuple(
            jnp.dot(lv, s_mat, preferred_element_type=jnp.float32,
                    precision=jax.lax.Precision.HIGHEST)
            for lv in pref
        )
        return _ta_mul(shifted, pref)

    s1, s2, s3, s4 = jax.lax.fori_loop(0, _ROUNDS, scan_body, sig)

    s1h = s1 * -0.5
    s2h = s2 * -0.5
    s3h = s3 * -0.5
    s1t = s1 * (1.0 / 3.0)
    s2t = s2 * (1.0 / 3.0)
    s1q = s1 * -0.25
    p2 = _tp(s1, s1)
    p3 = _tp(s1, s2) + _tp(s2, s1)
    q3 = _tp(s1, p2)
    l1 = s1
    l2 = s2 - 0.5 * p2
    l3 = s3 - 0.5 * p3 + (1.0 / 3.0) * q3
    l4_parts = []
    for m in range(_C):
        l4_parts.append(
            s4[m * 512:(m + 1) * 512]
            + s1h[m:m + 1] * s3
            + _tp(s2h[m * 8:(m + 1) * 8], s2)
            + _tp(s3h[m * 64:(m + 1) * 64], s1)
            + s1t[m:m + 1] * p3
            + _tp(s2t[m * 8:(m + 1) * 8], p2)
            + s1q[m:m + 1] * q3)
    l4 = jnp.concatenate(l4_parts, axis=0)

    m1 = jnp.mean(l1, axis=1, keepdims=True)
    m2 = jnp.mean(l2, axis=1, keepdims=True)
    m3 = jnp.mean(l3, axis=1, keepdims=True)
    m4 = jnp.mean(l4, axis=1, keepdims=True)

    dn = (((0,), (0,)), ((), ()))
    h = (jax.lax.dot_general(m1, w1_ref[0:8, :], dn,
                             preferred_element_type=jnp.float32)
         + jax.lax.dot_general(m2, w1_ref[8:72, :], dn,
                               preferred_element_type=jnp.float32)
         + jax.lax.dot_general(m3, w1_ref[72:584, :], dn,
                               preferred_element_type=jnp.float32)
         + jax.lax.dot_general(m4, w1_ref[584:4680, :], dn,
                               preferred_element_type=jnp.float32)
         + b1_ref[...])
    h = jnp.maximum(h, 0.0)
    out_ref[...] = (jnp.dot(h, w2_ref[...], preferred_element_type=jnp.float32)
                    + b2_ref[...])[None]


def kernel(x, W1, b1, W2, b2):
    B, T, Cin = x.shape
    C = Cin + 1
    N = T // _L
    H = W1.shape[1]

    t = jnp.linspace(0.0, 1.0, T, dtype=x.dtype)
    path = jnp.concatenate(
        [x, jnp.broadcast_to(t[None, :, None], (B, T, 1)).astype(x.dtype)],
        axis=-1)
    inc = jnp.diff(path, axis=1, prepend=jnp.zeros((B, 1, C), x.dtype))
    inc_t = inc.reshape(B, N, _L, C).transpose(0, 2, 3, 1)

    shifts = jnp.asarray(_SHIFTS)
    b1_2d = b1.reshape(1, H)
    b2_2d = b2.reshape(1, 1)

    out = pl.pallas_call(
        _blamem_kernel,
        grid=(B,),
        in_specs=[
            pl.BlockSpec((1, _L, C, N), lambda b: (b, 0, 0, 0)),
            pl.BlockSpec((_ROUNDS, _N, _N), lambda b: (0, 0, 0)),
            pl.BlockSpec(W1.shape, lambda b: (0, 0)),
            pl.BlockSpec((1, H), lambda b: (0, 0)),
            pl.BlockSpec(W2.shape, lambda b: (0, 0)),
            pl.BlockSpec((1, 1), lambda b: (0, 0)),
        ],
        out_specs=pl.BlockSpec((1, 1, 1), lambda b: (b, 0, 0)),
        out_shape=jax.ShapeDtypeStruct((B, 1, 1), jnp.float32),
        compiler_params=pltpu.CompilerParams(
            dimension_semantics=("arbitrary",),
            vmem_limit_bytes=56 * 1024 * 1024,
        ),
    )(inc_t, shifts, W1, b1_2d, W2, b2_2d)
    return out.reshape(B, 1)


# speedup vs baseline: 1.3507x; 1.3507x over previous
import numpy as np

import jax
import jax.numpy as jnp
from jax.experimental import pallas as pl
from jax.experimental.pallas import tpu as pltpu

_C = 8
_L = 16
_N = 128
_ROUNDS = 7


def _tp(a, b):
    A, n = a.shape
    Bd = b.shape[0]
    return (a[:, None, :] * b[None, :, :]).reshape(A * Bd, n)


def _exp1(d):
    e2 = _tp(d * 0.5, d)
    e3 = _tp(e2, d * (1.0 / 3.0))
    e4 = _tp(e3, d * 0.25)
    return (d, e2, e3, e4)


def _mul3(a, b):
    a1, a2, a3 = a[0], a[1], a[2]
    b1, b2, b3 = b[0], b[1], b[2]
    parts = []
    for m in range(_C):
        parts.append(a3[m * 64:(m + 1) * 64] + b3[m * 64:(m + 1) * 64]
                     + a1[m:m + 1] * b2
                     + _tp(a2[m * 8:(m + 1) * 8], b1))
    return jnp.concatenate(parts, axis=0)


def _mul4(a, b):
    a1, a2, a3, a4 = a
    b1, b2, b3, b4 = b
    parts = []
    for m in range(_C):
        parts.append(a4[m * 512:(m + 1) * 512] + b4[m * 512:(m + 1) * 512]
                     + a1[m:m + 1] * b3
                     + _tp(a2[m * 8:(m + 1) * 8], b2)
                     + _tp(a3[m * 64:(m + 1) * 64], b1))
    return jnp.concatenate(parts, axis=0)


def _ta_mul(a, b):
    c1 = a[0] + b[0]
    c2 = a[1] + b[1] + _tp(a[0], b[0])
    c3 = _mul3(a, b)
    c4 = _mul4(a, b)
    return (c1, c2, c3, c4)


def _chen_step(carry, d):
    a1, a2, a3, a4 = carry
    e2 = _tp(d * 0.5, d)
    e3 = _tp(e2, d * (1.0 / 3.0))
    dq = d * 0.25
    c1 = a1 + d
    c2 = a2 + e2 + _tp(a1, d)
    p3 = []
    p4 = []
    for m in range(_C):
        p3.append(a3[m * 64:(m + 1) * 64] + e3[m * 64:(m + 1) * 64]
                  + a1[m:m + 1] * e2
                  + _tp(a2[m * 8:(m + 1) * 8], d))
        p4.append(a4[m * 512:(m + 1) * 512]
                  + _tp(e3[m * 64:(m + 1) * 64], dq)
                  + a1[m:m + 1] * e3
                  + _tp(a2[m * 8:(m + 1) * 8], e2)
                  + _tp(a3[m * 64:(m + 1) * 64], d))
    return (c1, c2, jnp.concatenate(p3, axis=0), jnp.concatenate(p4, axis=0))


def _blamem_kernel(inc_ref, w1_ref, b1_ref, w2_ref, b2_ref, out_ref):
    carry0 = _exp1(inc_ref[0, 0])

    def chen_body(s, carry):
        d = inc_ref[0, s]
        return _chen_step(carry, d)

    sig = jax.lax.fori_loop(1, _L, chen_body, carry0)

    lane = jax.lax.broadcasted_iota(jnp.int32, (1, _N), 1)

    def scan_body(i, pref):
        d = jax.lax.shift_left(jnp.int32(1), i)
        maskf = (lane >= d).astype(jnp.float32)
        shifted = tuple(pltpu.roll(lv, d, 1) * maskf for lv in pref)
        return _ta_mul(shifted, pref)

    s1, s2, s3, s4 = jax.lax.fori_loop(0, _ROUNDS, scan_body, sig)

    s1h = s1 * -0.5
    s2h = s2 * -0.5
    s3h = s3 * -0.5
    s1t = s1 * (1.0 / 3.0)
    s2t = s2 * (1.0 / 3.0)
    s1q = s1 * -0.25
    p2 = _tp(s1, s1)
    p3 = _tp(s1, s2) + _tp(s2, s1)
    q3 = _tp(s1, p2)
    l1 = s1
    l2 = s2 - 0.5 * p2
    l3 = s3 - 0.5 * p3 + (1.0 / 3.0) * q3
    l4_parts = []
    for m in range(_C):
        l4_parts.append(
            s4[m * 512:(m + 1) * 512]
            + s1h[m:m + 1] * s3
            + _tp(s2h[m * 8:(m + 1) * 8], s2)
            + _tp(s3h[m * 64:(m + 1) * 64], s1)
            + s1t[m:m + 1] * p3
            + _tp(s2t[m * 8:(m + 1) * 8], p2)
            + s1q[m:m + 1] * q3)
    l4 = jnp.concatenate(l4_parts, axis=0)

    m1 = jnp.mean(l1, axis=1, keepdims=True)
    m2 = jnp.mean(l2, axis=1, keepdims=True)
    m3 = jnp.mean(l3, axis=1, keepdims=True)
    m4 = jnp.mean(l4, axis=1, keepdims=True)

    dn = (((0,), (0,)), ((), ()))
    h = (jax.lax.dot_general(m1, w1_ref[0:8, :], dn,
                             preferred_element_type=jnp.float32)
         + jax.lax.dot_general(m2, w1_ref[8:72, :], dn,
                               preferred_element_type=jnp.float32)
         + jax.lax.dot_general(m3, w1_ref[72:584, :], dn,
                               preferred_element_type=jnp.float32)
         + jax.lax.dot_general(m4, w1_ref[584:4680, :], dn,
                               preferred_element_type=jnp.float32)
         + b1_ref[...])
    h = jnp.maximum(h, 0.0)
    out_ref[...] = (jnp.dot(h, w2_ref[...], preferred_element_type=jnp.float32)
                    + b2_ref[...])[None]


def kernel(x, W1, b1, W2, b2):
    B, T, Cin = x.shape
    C = Cin + 1
    N = T // _L
    H = W1.shape[1]

    t = jnp.linspace(0.0, 1.0, T, dtype=x.dtype)
    path = jnp.concatenate(
        [x, jnp.broadcast_to(t[None, :, None], (B, T, 1)).astype(x.dtype)],
        axis=-1)
    inc = jnp.diff(path, axis=1, prepend=jnp.zeros((B, 1, C), x.dtype))
    inc_t = inc.reshape(B, N, _L, C).transpose(0, 2, 3, 1)

    b1_2d = b1.reshape(1, H)
    b2_2d = b2.reshape(1, 1)

    out = pl.pallas_call(
        _blamem_kernel,
        grid=(B,),
        in_specs=[
            pl.BlockSpec((1, _L, C, N), lambda b: (b, 0, 0, 0)),
            pl.BlockSpec(W1.shape, lambda b: (0, 0)),
            pl.BlockSpec((1, H), lambda b: (0, 0)),
            pl.BlockSpec(W2.shape, lambda b: (0, 0)),
            pl.BlockSpec((1, 1), lambda b: (0, 0)),
        ],
        out_specs=pl.BlockSpec((1, 1, 1), lambda b: (b, 0, 0)),
        out_shape=jax.ShapeDtypeStruct((B, 1, 1), jnp.float32),
        compiler_params=pltpu.CompilerParams(
            dimension_semantics=("arbitrary",),
            vmem_limit_bytes=56 * 1024 * 1024,
        ),
    )(inc_t, W1, b1_2d, W2, b2_2d)
    return out.reshape(B, 1)
